# Initial kernel scaffold; baseline (speedup 1.0000x reference)
import jax
import jax.numpy as jnp
from jax.experimental import pallas as pl


def kernel(img, erase_indices):
    raise NotImplementedError("write your pallas kernel here")


# speedup vs baseline: 4.6309x; 4.6309x over previous
import functools

import jax
import jax.numpy as jnp
from jax import lax
from jax.experimental import pallas as pl
from jax.experimental.pallas import tpu as pltpu
from jax.experimental.pallas import tpu_sc as plsc

_PATCH = 16
_CONST = 0.7
_C, _H, _W = 3, 512, 512
_NH, _NW = _H // _PATCH, _W // _PATCH
_LANES = 16


def _body(n_idx, img_hbm, idx_hbm, out_hbm, idx_v, buf_v):
    ci = lax.axis_index("c")
    si = lax.axis_index("s")
    wid = si * 2 + ci

    pltpu.sync_copy(idx_hbm, idx_v.at[pl.ds(0, n_idx)])

    nk = (n_idx + _LANES - 1) // _LANES
    acc = jnp.zeros((_LANES,), jnp.int32)
    for k in range(nk):
        p = idx_v[pl.ds(k * _LANES, _LANES)]
        lane = lax.iota(jnp.int32, _LANES) + (k * _LANES)
        valid = lane < n_idx
        m = jnp.logical_and(valid, jnp.right_shift(p, 5) == wid)
        pw = jnp.bitwise_and(p, _NW - 1)
        acc = jnp.bitwise_or(acc, jnp.where(m, jnp.left_shift(1, pw), 0))
    lanes = [acc[l] for l in range(_LANES)]
    while len(lanes) > 1:
        lanes = [
            jnp.bitwise_or(lanes[i], lanes[i + 1]) if i + 1 < len(lanes)
            else lanes[i]
            for i in range(0, len(lanes), 2)
        ]
    bits = lanes[0]

    cvec = jnp.full((_LANES,), _CONST, jnp.float32)
    for c in range(_C):
        pltpu.sync_copy(img_hbm.at[c, pl.ds(wid * _PATCH, _PATCH), :], buf_v)
        for pw in range(_NW):
            @pl.when(jnp.bitwise_and(jnp.right_shift(bits, pw), 1) != 0)
            def _erase(pw=pw):
                for r in range(_PATCH):
                    buf_v[r, pl.ds(pw * _PATCH, _PATCH)] = cvec
        pltpu.sync_copy(buf_v, out_hbm.at[c, pl.ds(wid * _PATCH, _PATCH), :])


def kernel(img, erase_indices):
    n_idx = erase_indices.shape[0]
    n_pad = ((n_idx + _LANES - 1) // _LANES) * _LANES
    mesh = plsc.VectorSubcoreMesh(
        core_axis_name="c", subcore_axis_name="s", num_cores=2, num_subcores=16
    )
    run = functools.partial(
        pl.kernel,
        out_type=jax.ShapeDtypeStruct((_C, _H, _W), jnp.float32),
        mesh=mesh,
        scratch_types=[
            pltpu.VMEM((n_pad,), jnp.int32),
            pltpu.VMEM((_PATCH, _W), jnp.float32),
        ],
    )(functools.partial(_body, n_idx))
    return run(img, erase_indices.astype(jnp.int32))


# speedup vs baseline: 5.0520x; 1.0909x over previous
import functools

import jax
import jax.numpy as jnp
from jax import lax
from jax.experimental import pallas as pl
from jax.experimental.pallas import tpu as pltpu
from jax.experimental.pallas import tpu_sc as plsc

_PATCH = 16
_CONST = 0.7
_C, _H, _W = 3, 512, 512
_NH, _NW = _H // _PATCH, _W // _PATCH
_LANES = 16


def _body(n_idx, img_hbm, idx_hbm, out_hbm, idx_v, b0, b1, b2,
          si0, si1, si2, so0, so1, so2):
    ci = lax.axis_index("c")
    si = lax.axis_index("s")
    wid = si * 2 + ci

    bufs = (b0, b1, b2)
    isems = (si0, si1, si2)
    osems = (so0, so1, so2)
    rows = pl.ds(wid * _PATCH, _PATCH)

    loads = [
        pltpu.async_copy(img_hbm.at[c, rows, :], bufs[c], isems[c])
        for c in range(_C)
    ]
    pltpu.sync_copy(idx_hbm, idx_v.at[pl.ds(0, n_idx)])

    nk = (n_idx + _LANES - 1) // _LANES
    acc = jnp.zeros((_LANES,), jnp.int32)
    for k in range(nk):
        p = idx_v[pl.ds(k * _LANES, _LANES)]
        lane = lax.iota(jnp.int32, _LANES) + (k * _LANES)
        valid = lane < n_idx
        m = jnp.logical_and(valid, jnp.right_shift(p, 5) == wid)
        pw = jnp.bitwise_and(p, _NW - 1)
        acc = jnp.bitwise_or(acc, jnp.where(m, jnp.left_shift(1, pw), 0))
    lanes = [acc[l] for l in range(_LANES)]
    while len(lanes) > 1:
        lanes = [
            jnp.bitwise_or(lanes[i], lanes[i + 1]) if i + 1 < len(lanes)
            else lanes[i]
            for i in range(0, len(lanes), 2)
        ]
    bits = lanes[0]

    cvec = jnp.full((_LANES,), _CONST, jnp.float32)
    stores = []
    for c in range(_C):
        loads[c].wait()
        for pw in range(_NW):
            @pl.when(jnp.bitwise_and(jnp.right_shift(bits, pw), 1) != 0)
            def _erase(pw=pw, c=c):
                for r in range(_PATCH):
                    bufs[c][r, pl.ds(pw * _PATCH, _PATCH)] = cvec
        stores.append(pltpu.async_copy(bufs[c], out_hbm.at[c, rows, :], osems[c]))
    for st in stores:
        st.wait()


def kernel(img, erase_indices):
    n_idx = erase_indices.shape[0]
    n_pad = ((n_idx + _LANES - 1) // _LANES) * _LANES
    mesh = plsc.VectorSubcoreMesh(
        core_axis_name="c", subcore_axis_name="s", num_cores=2, num_subcores=16
    )
    run = functools.partial(
        pl.kernel,
        out_type=jax.ShapeDtypeStruct((_C, _H, _W), jnp.float32),
        mesh=mesh,
        scratch_types=[
            pltpu.VMEM((n_pad,), jnp.int32),
            pltpu.VMEM((_PATCH, _W), jnp.float32),
            pltpu.VMEM((_PATCH, _W), jnp.float32),
            pltpu.VMEM((_PATCH, _W), jnp.float32),
            pltpu.SemaphoreType.DMA,
            pltpu.SemaphoreType.DMA,
            pltpu.SemaphoreType.DMA,
            pltpu.SemaphoreType.DMA,
            pltpu.SemaphoreType.DMA,
            pltpu.SemaphoreType.DMA,
        ],
    )(functools.partial(_body, n_idx))
    return run(img, erase_indices.astype(jnp.int32))


# speedup vs baseline: 5.5380x; 1.0962x over previous
import functools

import jax
import jax.numpy as jnp
from jax import lax
from jax.experimental import pallas as pl
from jax.experimental.pallas import tpu as pltpu
from jax.experimental.pallas import tpu_sc as plsc

_PATCH = 16
_CONST = 0.7
_C, _H, _W = 3, 512, 512
_NH, _NW = _H // _PATCH, _W // _PATCH
_LANES = 16


def _body(n_idx, img_hbm, idx_hbm, out_hbm, idx_v, buf, si0, si1, si2,
          so0, so1, so2):
    ci = lax.axis_index("c")
    si = lax.axis_index("s")
    wid = si * 2 + ci
    rows = pl.ds(wid * _PATCH, _PATCH)
    isems = (si0, si1, si2)
    osems = (so0, so1, so2)

    loads = [
        pltpu.async_copy(img_hbm.at[c, rows, :], buf.at[c], isems[c])
        for c in range(_C)
    ]
    pltpu.sync_copy(idx_hbm, idx_v.at[pl.ds(0, n_idx)])

    nk = (n_idx + _LANES - 1) // _LANES
    iota = lax.iota(jnp.int32, _LANES)

    def bit_step(k, acc):
        p = idx_v[pl.ds(k * _LANES, _LANES)]
        valid = (iota + k * _LANES) < n_idx
        m = jnp.logical_and(valid, jnp.right_shift(p, 5) == wid)
        pw = jnp.bitwise_and(p, _NW - 1)
        return jnp.bitwise_or(acc, jnp.where(m, jnp.left_shift(1, pw), 0))

    acc = lax.fori_loop(0, nk, bit_step, jnp.zeros((_LANES,), jnp.int32))
    lanes = [acc[l] for l in range(_LANES)]
    while len(lanes) > 1:
        lanes = [
            jnp.bitwise_or(lanes[i], lanes[i + 1]) if i + 1 < len(lanes)
            else lanes[i]
            for i in range(0, len(lanes), 2)
        ]
    bits = lanes[0]

    for ld in loads:
        ld.wait()

    cvec = jnp.full((_LANES,), _CONST, jnp.float32)
    for j in range(_NW):
        @pl.when(jnp.bitwise_and(jnp.right_shift(bits, j), 1) != 0)
        def _erase(j=j):
            def row_step(r, carry):
                for c in range(_C):
                    buf[c, r, pl.ds(j * _PATCH, _PATCH)] = cvec
                return carry
            lax.fori_loop(0, _PATCH, row_step, 0)

    stores = [
        pltpu.async_copy(buf.at[c], out_hbm.at[c, rows, :], osems[c])
        for c in range(_C)
    ]
    for st in stores:
        st.wait()


def kernel(img, erase_indices):
    n_idx = erase_indices.shape[0]
    n_pad = ((n_idx + _LANES - 1) // _LANES) * _LANES
    mesh = plsc.VectorSubcoreMesh(
        core_axis_name="c", subcore_axis_name="s", num_cores=2, num_subcores=16
    )
    run = functools.partial(
        pl.kernel,
        out_type=jax.ShapeDtypeStruct((_C, _H, _W), jnp.float32),
        mesh=mesh,
        scratch_types=[
            pltpu.VMEM((n_pad,), jnp.int32),
            pltpu.VMEM((_C, _PATCH, _W), jnp.float32),
            pltpu.SemaphoreType.DMA,
            pltpu.SemaphoreType.DMA,
            pltpu.SemaphoreType.DMA,
            pltpu.SemaphoreType.DMA,
            pltpu.SemaphoreType.DMA,
            pltpu.SemaphoreType.DMA,
        ],
    )(functools.partial(_body, n_idx))
    return run(img, erase_indices.astype(jnp.int32))


# speedup vs baseline: 5.5621x; 1.0043x over previous
import functools

import jax
import jax.numpy as jnp
from jax import lax
from jax.experimental import pallas as pl
from jax.experimental.pallas import tpu as pltpu
from jax.experimental.pallas import tpu_sc as plsc

_PATCH = 16
_CONST = 0.7
_C, _H, _W = 3, 512, 512
_NH, _NW = _H // _PATCH, _W // _PATCH
_LANES = 16


def _body(n_idx, img_hbm, idx_hbm, out_hbm, idx_v, buf, si0, si1, si2,
          so0, so1, so2):
    ci = lax.axis_index("c")
    si = lax.axis_index("s")
    wid = si * 2 + ci
    rows = pl.ds(wid * _PATCH, _PATCH)
    isems = (si0, si1, si2)
    osems = (so0, so1, so2)

    loads = [pltpu.async_copy(img_hbm.at[:, rows, :], buf, si0)]
    pltpu.sync_copy(idx_hbm, idx_v.at[pl.ds(0, n_idx)])

    nk = (n_idx + _LANES - 1) // _LANES
    iota = lax.iota(jnp.int32, _LANES)

    def bit_step(k, acc):
        p = idx_v[pl.ds(k * _LANES, _LANES)]
        valid = (iota + k * _LANES) < n_idx
        m = jnp.logical_and(valid, jnp.right_shift(p, 5) == wid)
        pw = jnp.bitwise_and(p, _NW - 1)
        return jnp.bitwise_or(acc, jnp.where(m, jnp.left_shift(1, pw), 0))

    acc = lax.fori_loop(0, nk, bit_step, jnp.zeros((_LANES,), jnp.int32))
    lanes = [acc[l] for l in range(_LANES)]
    while len(lanes) > 1:
        lanes = [
            jnp.bitwise_or(lanes[i], lanes[i + 1]) if i + 1 < len(lanes)
            else lanes[i]
            for i in range(0, len(lanes), 2)
        ]
    bits = lanes[0]

    for ld in loads:
        ld.wait()

    cvec = jnp.full((_LANES,), _CONST, jnp.float32)
    for j in range(_NW):
        @pl.when(jnp.bitwise_and(jnp.right_shift(bits, j), 1) != 0)
        def _erase(j=j):
            def row_step(r, carry):
                for c in range(_C):
                    buf[c, r, pl.ds(j * _PATCH, _PATCH)] = cvec
                return carry
            lax.fori_loop(0, _PATCH, row_step, 0)

    stores = [pltpu.async_copy(buf, out_hbm.at[:, rows, :], so0)]
    for st in stores:
        st.wait()


def kernel(img, erase_indices):
    n_idx = erase_indices.shape[0]
    n_pad = ((n_idx + _LANES - 1) // _LANES) * _LANES
    mesh = plsc.VectorSubcoreMesh(
        core_axis_name="c", subcore_axis_name="s", num_cores=2, num_subcores=16
    )
    run = functools.partial(
        pl.kernel,
        out_type=jax.ShapeDtypeStruct((_C, _H, _W), jnp.float32),
        mesh=mesh,
        scratch_types=[
            pltpu.VMEM((n_pad,), jnp.int32),
            pltpu.VMEM((_C, _PATCH, _W), jnp.float32),
            pltpu.SemaphoreType.DMA,
            pltpu.SemaphoreType.DMA,
            pltpu.SemaphoreType.DMA,
            pltpu.SemaphoreType.DMA,
            pltpu.SemaphoreType.DMA,
            pltpu.SemaphoreType.DMA,
        ],
    )(functools.partial(_body, n_idx))
    return run(img, erase_indices.astype(jnp.int32))
